# Initial kernel scaffold; baseline (speedup 1.0000x reference)
import jax
import jax.numpy as jnp
from jax.experimental import pallas as pl


def kernel(x, w_imp1, b_imp1, w_imp2, b_imp2, Wq, bq, Wk, bk, Wv, bv, Wo, bo, ln_g, ln_b, lra_down, lra_up, lra_alpha, dw_w, dw_b, se_dw, se_db, se_uw, se_ub, ula_alpha):
    raise NotImplementedError("write your pallas kernel here")


# speedup vs baseline: 1.2969x; 1.2969x over previous
import functools

import jax
import jax.numpy as jnp
from jax.experimental import pallas as pl
from jax.experimental.pallas import tpu as pltpu

F32 = jnp.float32


def _gelu(v):
    return 0.5 * v * (1.0 + jax.lax.erf(v * (2.0 ** -0.5)))


def _sigmoid(v):
    return jax.nn.sigmoid(v)


def _shift_flat(a, d):
    if d == 0:
        return a
    r = a.shape[0]
    z = jnp.zeros((r, abs(d)), a.dtype)
    if d > 0:
        return jnp.concatenate([a[:, d:], z], axis=1)
    return jnp.concatenate([z, a[:, :d]], axis=1)


def _hmask(a, dx, col, wdim):
    if dx > 0:
        return jnp.where(col < wdim - dx, a, 0.0)
    if dx < 0:
        return jnp.where(col >= -dx, a, 0.0)
    return a


def _img_shift(a, dy, dx, col, wdim):
    return _hmask(_shift_flat(a, dy * wdim + dx), dx, col, wdim)


def _sepconv(img, ucoefs, vcoefs, col, wdim):
    s = len(vcoefs) // 2
    tmp = None
    for j, c in enumerate(vcoefs):
        if c == 0:
            continue
        t = _hmask(_shift_flat(img, j - s), j - s, col, wdim) * c
        tmp = t if tmp is None else tmp + t
    out = None
    for i, c in enumerate(ucoefs):
        if c == 0:
            continue
        t = _shift_flat(tmp, (i - s) * wdim) * c
        out = t if out is None else out + t
    return out


def _excl_cumsum(x, rows, lanes):
    xr = x.reshape(rows, lanes)
    i0 = jax.lax.broadcasted_iota(jnp.int32, (lanes, lanes), 0)
    i1 = jax.lax.broadcasted_iota(jnp.int32, (lanes, lanes), 1)
    m = (i0 < i1).astype(F32)
    inrow = jax.lax.dot_general(xr, m, (((1,), (0,)), ((), ())),
                                preferred_element_type=F32)
    tot = jnp.sum(xr, axis=1, keepdims=True)
    r0 = jax.lax.broadcasted_iota(jnp.int32, (rows, rows), 0)
    r1 = jax.lax.broadcasted_iota(jnp.int32, (rows, rows), 1)
    lm = (r1 < r0).astype(F32)
    offs = jax.lax.dot_general(lm, tot, (((1,), (0,)), ((), ())),
                               preferred_element_type=F32)
    return (inrow + offs).reshape(1, rows * lanes)


def _body(xe_ref, imp_ref,
          wq_ref, bq_ref, wk_ref, bk_ref, wv_ref, bv_ref, wo_ref, bo_ref,
          lng_ref, lnb_ref, ld_ref, lu_ref, la_ref,
          dw9_ref, dwb_ref, sdw_ref, sdb_ref, suw_ref, sub_ref, ua_ref,
          out_ref, *, C, H, W, K, nh):
    N = H * W
    hd = C // nh
    xe = xe_ref[0]
    imp = imp_ref[0]
    col = jax.lax.broadcasted_iota(jnp.int32, (1, N), 1) % W

    key = jax.lax.bitcast_convert_type(imp, jnp.int32)

    def bs_body(_, carry):
        lo, hi = carry
        mid = lo + (hi - lo) // 2
        cnt = jnp.sum((key > mid).astype(F32))
        small = cnt < K
        return (jnp.where(small, lo, mid), jnp.where(small, mid, hi))

    lo0 = jnp.int32(-1)
    hi0 = jnp.int32(0x40000000)
    _, t = jax.lax.fori_loop(0, 31, bs_body, (lo0, hi0))
    gt = key > t
    n_gt = jnp.sum(gt.astype(F32))
    tie = key == t
    tie_excl = _excl_cumsum(tie.astype(F32), 32, N // 32)
    sel = jnp.logical_or(gt, jnp.logical_and(tie, tie_excl < (K - n_gt)))

    slot = jnp.where(sel, _excl_cumsum(sel.astype(F32), 32, N // 32),
                     -1.0).astype(jnp.int32)
    jio = jax.lax.broadcasted_iota(jnp.int32, (K, N), 0)
    P = (jio == slot).astype(F32)

    xs = jax.lax.dot_general(xe, P, (((1,), (1,)), ((), ())),
                             preferred_element_type=F32)

    q = jnp.dot(wq_ref[...], xs, preferred_element_type=F32) + bq_ref[...]
    kk = jnp.dot(wk_ref[...], xs, preferred_element_type=F32) + bk_ref[...]
    vv = jnp.dot(wv_ref[...], xs, preferred_element_type=F32) + bv_ref[...]
    scale = float(hd) ** -0.5
    outs = []
    for h in range(nh):
        qh = q[h * hd:(h + 1) * hd, :]
        kh = kk[h * hd:(h + 1) * hd, :]
        vh = vv[h * hd:(h + 1) * hd, :]
        s = jax.lax.dot_general(qh, kh, (((0,), (0,)), ((), ())),
                                preferred_element_type=F32) * scale
        s = s - jnp.max(s, axis=1, keepdims=True)
        e = jnp.exp(s)
        a = e / jnp.sum(e, axis=1, keepdims=True)
        outs.append(jax.lax.dot_general(vh, a, (((1,), (1,)), ((), ())),
                                        preferred_element_type=F32))
    att = jnp.concatenate(outs, axis=0)
    os_ = jnp.dot(wo_ref[...], att, preferred_element_type=F32) + bo_ref[...]
    pre = os_ + xs
    pmu = jnp.mean(pre, axis=0, keepdims=True)
    pvar = jnp.mean((pre - pmu) ** 2, axis=0, keepdims=True)
    enh = (pre - pmu) / jnp.sqrt(pvar + 1e-5) * lng_ref[...] + lnb_ref[...]

    scat = jax.lax.dot_general(enh, P, (((1,), (0,)), ((), ())),
                               preferred_element_type=F32)
    y = jnp.where(sel, scat, xe)

    down = _gelu(jnp.dot(ld_ref[...], y, preferred_element_type=F32))
    delta = jnp.dot(lu_ref[...], down, preferred_element_type=F32)
    y = y + la_ref[0, 0] * delta

    sp = None
    for i in range(3):
        for j in range(3):
            tp = _img_shift(y, i - 1, j - 1, col, W) * dw9_ref[:, 3 * i + j:3 * i + j + 1]
            sp = tp if sp is None else sp + tp
    sp = _gelu(sp + dwb_ref[...])

    gap = jnp.mean(y, axis=1, keepdims=True)
    z = _gelu(jnp.dot(sdw_ref[...], gap, preferred_element_type=F32)
              + sdb_ref[...])
    ch = _sigmoid(jnp.dot(suw_ref[...], z, preferred_element_type=F32)
                  + sub_ref[...])
    out_ref[0] = y + ua_ref[0, 0] * (sp * ch)


def _conv2d(x, w, pad, groups=1):
    return jax.lax.conv_general_dilated(
        x, w, (1, 1), [(pad, pad), (pad, pad)],
        dimension_numbers=('NCHW', 'OIHW', 'NCHW'), feature_group_count=groups)


_S3 = ((-1., 0., 1.), (-2., 0., 2.), (-1., 0., 1.))
_S5 = ((-1., -2., 0., 2., 1.), (-4., -8., 0., 8., 4.), (-6., -12., 0., 12., 6.),
       (-4., -8., 0., 8., 4.), (-1., -2., 0., 2., 1.))
_S7 = ((-1., -4., -5., 0., 5., 4., 1.), (-6., -24., -30., 0., 30., 24., 6.),
       (-15., -60., -75., 0., 75., 60., 15.),
       (-20., -80., -100., 0., 100., 80., 20.),
       (-15., -60., -75., 0., 75., 60., 15.),
       (-6., -24., -30., 0., 30., 24., 6.), (-1., -4., -5., 0., 5., 4., 1.))
_LAP = ((0., 1., 0.), (1., -4., 1.), (0., 1., 0.))


def _scores(x, w_imp1, b_imp1, w_imp2, b_imp2):
    xg = x.mean(axis=1, keepdims=True)
    maps = []
    for kern in (_S3, _S5, _S7):
        ka = jnp.asarray(kern, F32)
        gx = _conv2d(xg, ka[None, None], ka.shape[0] // 2)
        gy = _conv2d(xg, ka.T[None, None], ka.shape[0] // 2)
        maps.append(jnp.sqrt(gx ** 2 + gy ** 2 + 1e-06))
    maps.append(jnp.abs(_conv2d(xg, jnp.asarray(_LAP, F32)[None, None], 1)))
    comb = jnp.concatenate(maps, axis=1).mean(axis=1, keepdims=True)
    mu = comb.mean(axis=(2, 3), keepdims=True)
    sd = jnp.std(comb, axis=(2, 3), keepdims=True, ddof=1) + 1e-06
    bm = jax.nn.sigmoid(5.0 * (comb - mu) / sd - 2.5)
    xe = x * (1.0 + bm)
    h1 = jax.nn.gelu(_conv2d(xe, w_imp1, 0) + b_imp1[None, :, None, None],
                     approximate=False)
    imp = jax.nn.sigmoid(_conv2d(h1, w_imp2, 0) + b_imp2[None, :, None, None])
    return xe, imp + 0.5 * bm


def kernel(x, w_imp1, b_imp1, w_imp2, b_imp2, Wq, bq, Wk, bk, Wv, bv, Wo, bo,
           ln_g, ln_b, lra_down, lra_up, lra_alpha, dw_w, dw_b,
           se_dw, se_db, se_uw, se_ub, ula_alpha):
    B, C, H, W = x.shape
    N = H * W
    K = max(N // 4, 1)
    nh = 8
    Cr = se_dw.shape[0]

    xe, imp = _scores(x, w_imp1, b_imp1, w_imp2, b_imp2)
    xe3 = xe.reshape(B, C, N)
    imp3 = imp.reshape(B, 1, N)
    args = [
        Wq, bq.reshape(C, 1), Wk, bk.reshape(C, 1), Wv, bv.reshape(C, 1),
        Wo, bo.reshape(C, 1), ln_g.reshape(C, 1), ln_b.reshape(C, 1),
        lra_down, lra_up, lra_alpha.reshape(1, 1),
        dw_w.reshape(C, 9), dw_b.reshape(C, 1),
        se_dw, se_db.reshape(Cr, 1), se_uw, se_ub.reshape(C, 1),
        ula_alpha.reshape(1, 1),
    ]

    in_specs = [pl.BlockSpec((1, C, N), lambda b: (b, 0, 0)),
                pl.BlockSpec((1, 1, N), lambda b: (b, 0, 0))]
    for a in args:
        sh = a.shape
        in_specs.append(pl.BlockSpec(sh, lambda b, _s=sh: (0,) * len(_s)))

    y = pl.pallas_call(
        functools.partial(_body, C=C, H=H, W=W, K=K, nh=nh),
        grid=(B,),
        in_specs=in_specs,
        out_specs=pl.BlockSpec((1, C, N), lambda b: (b, 0, 0)),
        out_shape=jax.ShapeDtypeStruct((B, C, N), F32),
        compiler_params=pltpu.CompilerParams(
            dimension_semantics=("arbitrary",),
            vmem_limit_bytes=100 * 1024 * 1024,
        ),
    )(xe3, imp3, *args)
    return y.reshape(B, C, H, W)
